# Initial kernel scaffold; baseline (speedup 1.0000x reference)
import jax
import jax.numpy as jnp
from jax.experimental import pallas as pl


def kernel(lbls, no_lbl_idx, knn_sc, knn_fc):
    raise NotImplementedError("write your pallas kernel here")


# speedup vs baseline: 3.6920x; 3.6920x over previous
import functools

import jax
import jax.numpy as jnp
from jax import lax
from jax.experimental import pallas as pl
from jax.experimental.pallas import tpu as pltpu
from jax.experimental.pallas import tpu_sc as plsc

N = 10000
E = 320000
D = 128
NC = 2
NS = 16
K = 80
NP = 10240
EPW = E // NS
NCHUNK = EPW // K
RPW = NP // NS

_mesh = plsc.VectorSubcoreMesh(core_axis_name="c", subcore_axis_name="s")


@functools.partial(
    pl.kernel,
    out_type=(
        jax.ShapeDtypeStruct((NC, NP, D), jnp.float32),
        jax.ShapeDtypeStruct((NC, NP, D), jnp.float32),
    ),
    mesh=_mesh,
    scratch_types=(
        pltpu.VMEM_SHARED((NP, D), jnp.float32),
        pltpu.VMEM((K,), jnp.int32),
        pltpu.VMEM((K,), jnp.int32),
        pltpu.VMEM((K, D), jnp.float32),
        pltpu.SemaphoreType.DMA,
    ),
)
def _segment_sums(srcs, dsts, lbls, zsum, ones_in,
                  out_sums, out_cnts,
                  acc_sh, src_v, dst_v, rows_v, sem):
    c = lax.axis_index("c")
    s = lax.axis_index("s")
    r0 = pl.multiple_of(s * RPW, 8)

    def zero_acc():
        pltpu.sync_copy(zsum, rows_v)
        for j in range(RPW // K):
            pltpu.sync_copy(rows_v, acc_sh.at[pl.ds(r0 + j * K, K)])

    def writeout(dst_hbm):
        for j in range(RPW // K):
            pltpu.sync_copy(acc_sh.at[pl.ds(r0 + j * K, K)], rows_v)
            pltpu.sync_copy(rows_v, dst_hbm.at[c, pl.ds(r0 + j * K, K)])

    zero_acc()
    plsc.subcore_barrier()

    def chunk_a(g, carry):
        off = pl.multiple_of(c * E + s * EPW + g * K, 8)
        pltpu.sync_copy(srcs.at[pl.ds(off, K)], src_v)
        pltpu.sync_copy(dsts.at[pl.ds(off, K)], dst_v)
        pltpu.async_copy(lbls.at[src_v], rows_v, sem).wait()
        pltpu.sync_copy(rows_v, acc_sh.at[dst_v], add=True)
        return carry

    lax.fori_loop(0, NCHUNK, chunk_a, 0)
    plsc.subcore_barrier()
    writeout(out_sums)
    plsc.subcore_barrier()

    zero_acc()
    pltpu.sync_copy(ones_in, rows_v)
    plsc.subcore_barrier()

    def chunk_b(g, carry):
        off = pl.multiple_of(c * E + s * EPW + g * K, 8)
        pltpu.sync_copy(dsts.at[pl.ds(off, K)], dst_v)
        pltpu.sync_copy(rows_v, acc_sh.at[dst_v], add=True)
        return carry

    lax.fori_loop(0, NCHUNK, chunk_b, 0)
    plsc.subcore_barrier()
    writeout(out_cnts)


_BR = 1000


def _combine_body(lbls_ref, mask_ref, s1_ref, c1_ref, s2_ref, c2_ref, o_ref):
    c1 = jnp.maximum(c1_ref[:, 0:1], 1.0)
    c2 = jnp.maximum(c2_ref[:, 0:1], 1.0)
    mean = (s1_ref[...] / c1 + s2_ref[...] / c2) * 0.5
    o_ref[...] = jnp.where(mask_ref[...] > 0, mean, lbls_ref[...])


def kernel(lbls, no_lbl_idx, knn_sc, knn_fc):
    srcs = jnp.concatenate([knn_sc[0], knn_fc[0]])
    dsts = jnp.concatenate([knn_sc[1], knn_fc[1]])
    zsum = jnp.zeros((K, D), jnp.float32)
    ones = jnp.ones((K, D), jnp.float32)
    sums, cnts = _segment_sums(srcs, dsts, lbls, zsum, ones)

    mask2d = no_lbl_idx.astype(jnp.int32).reshape(N, 1)
    return pl.pallas_call(
        _combine_body,
        out_shape=jax.ShapeDtypeStruct((N, D), jnp.float32),
        grid=(N // _BR,),
        in_specs=[
            pl.BlockSpec((_BR, D), lambda i: (i, 0)),
            pl.BlockSpec((_BR, 1), lambda i: (i, 0)),
            pl.BlockSpec((_BR, D), lambda i: (i, 0)),
            pl.BlockSpec((_BR, D), lambda i: (i, 0)),
            pl.BlockSpec((_BR, D), lambda i: (i, 0)),
            pl.BlockSpec((_BR, D), lambda i: (i, 0)),
        ],
        out_specs=pl.BlockSpec((_BR, D), lambda i: (i, 0)),
    )(lbls, mask2d, sums[0], cnts[0], sums[1], cnts[1])
